# baseline (speedup 1.0000x reference)
import functools

import jax
import jax.numpy as jnp
from jax import lax
from jax.experimental import pallas as pl
from jax.experimental.pallas import tpu as pltpu
from jax.experimental.pallas import tpu_sc as plsc

_VOCAB = 1000000
_VOCAB_PAD = 1000064
_EMB = 16
_B = 16384
_NF = 9
_NC_FEAT = 3
_LANES = 16
_CHL = 1664
_TC_GRID = _VOCAB_PAD // _CHL

_info = plsc.get_sparse_core_info()
_NW = _info.num_cores * _info.num_subcores
_BPW = _B // _NW
_CHUNKS = _BPW // _LANES

_mesh = plsc.VectorSubcoreMesh(core_axis_name="c", subcore_axis_name="s")


_N_CHUNKS = _VOCAB_PAD // _CHL
_MAX_CHUNKS_PER_W = -(-_N_CHUNKS // _NW)


@functools.partial(
    pl.kernel,
    mesh=_mesh,
    out_type=jax.ShapeDtypeStruct((_VOCAB_PAD, _EMB), jnp.float32),
    compiler_params=pltpu.CompilerParams(
        needs_layout_passes=False, use_tc_tiling_on_sc=False),
    scratch_types=[
        pltpu.VMEM((_EMB, _CHL), jnp.float32),
        pltpu.VMEM((_CHL, _EMB), jnp.float32),
    ],
)
def _relayout_sc(wt_hbm, out_hbm, in_v, out_v):
    wid = lax.axis_index("s") * _info.num_cores + lax.axis_index("c")
    lane = lax.iota(jnp.int32, _LANES)

    def chunk_body(i, _):
        cid = i * _NW + wid

        @pl.when(cid < _N_CHUNKS)
        def _():
            base = cid * _CHL
            pltpu.sync_copy(wt_hbm.at[:, pl.ds(base, _CHL)], in_v)

            def group_body(g, _):
                c0 = g * _LANES
                for r in range(_LANES):
                    col = plsc.load_gather(
                        in_v, [lane, jnp.full((_LANES,), c0 + r, jnp.int32)])
                    out_v[c0 + r] = col
                return ()

            lax.fori_loop(0, _CHL // _LANES, group_body, (), unroll=False)
            pltpu.sync_copy(out_v, out_hbm.at[pl.ds(base, _CHL), :])
        return ()

    lax.fori_loop(0, _MAX_CHUNKS_PER_W, chunk_body, (), unroll=False)


def _relayout(w_t):
    w_p = jnp.pad(w_t, ((0, 0), (0, _VOCAB_PAD - _VOCAB)))
    return _relayout_sc(w_p)


@functools.partial(
    pl.kernel,
    mesh=_mesh,
    out_type=jax.ShapeDtypeStruct((_B,), jnp.float32),
    compiler_params=pltpu.CompilerParams(
        needs_layout_passes=False, use_tc_tiling_on_sc=False),
    scratch_types=(
        [pltpu.VMEM((_BPW,), jnp.int32) for _ in range(_NF)]
        + [pltpu.VMEM((_NF, _BPW, _EMB), jnp.float32)]
        + [pltpu.VMEM((_BPW,), jnp.float32) for _ in range(_NF)]
        + [pltpu.VMEM((_BPW,), jnp.float32) for _ in range(_NC_FEAT)]
        + [
            pltpu.VMEM((_BPW,), jnp.float32),
            pltpu.VMEM((_LANES,), jnp.float32),
            pltpu.VMEM((_LANES, _LANES), jnp.float32),
            pltpu.SemaphoreType.DMA,
        ]
    ),
)
def _fm_sc(idx_hbm, cont_hbm, w_hbm, l_hbm, bias_hbm, out_hbm, *scratch):
    idx_vs = scratch[:_NF]
    rows_v = scratch[_NF]
    lrows_vs = scratch[_NF + 1:2 * _NF + 1]
    cont_vs = scratch[2 * _NF + 1:2 * _NF + 1 + _NC_FEAT]
    out_v, bias_v, tbuf, sem = scratch[2 * _NF + 1 + _NC_FEAT:]

    wid = lax.axis_index("s") * _info.num_cores + lax.axis_index("c")
    base = wid * _BPW

    for j in range(_NF):
        pltpu.sync_copy(idx_hbm.at[pl.ds(j * _B + base, _BPW)], idx_vs[j])
    for k in range(_NC_FEAT):
        pltpu.sync_copy(cont_hbm.at[pl.ds(k * _B + base, _BPW)], cont_vs[k])
    pltpu.sync_copy(bias_hbm, bias_v)

    copies = []
    for j in range(_NF):
        copies.append(pltpu.async_copy(w_hbm.at[idx_vs[j]], rows_v.at[j], sem))
        copies.append(pltpu.async_copy(l_hbm.at[idx_vs[j]], lrows_vs[j], sem))
    for c in copies:
        c.wait()

    lane = lax.iota(jnp.int32, _LANES)
    bias_vec = bias_v[...]

    def chunk_body(c, _):
        row0 = c * _LANES
        for r in range(_LANES):
            row = row0 + r
            e = rows_v[0, row]
            s = e
            ss = e * e
            for j in range(1, _NF):
                e = rows_v[j, row]
                s = s + e
                ss = ss + e * e
            tbuf[r] = s * s - ss
        res = jnp.zeros((_LANES,), jnp.float32)
        for dd in range(_LANES):
            col = plsc.load_gather(
                tbuf, [lane, jnp.full((_LANES,), dd, jnp.int32)])
            res = res + col
        lin = bias_vec
        for j in range(_NF - _NC_FEAT):
            lin = lin + lrows_vs[j][pl.ds(row0, _LANES)]
        for k in range(_NC_FEAT):
            lin = lin + (lrows_vs[_NF - _NC_FEAT + k][pl.ds(row0, _LANES)]
                         * cont_vs[k][pl.ds(row0, _LANES)])
        z = lin + 0.5 * res
        out_v[pl.ds(row0, _LANES)] = 1.0 / (1.0 + jnp.exp(-z))
        return ()

    lax.fori_loop(0, _CHUNKS, chunk_body, (), unroll=False)

    pltpu.sync_copy(out_v, out_hbm.at[pl.ds(base, _BPW)])


def kernel(x, W, L, bias):
    idx = x[:, :_NF].astype(jnp.int32).T.reshape(_NF * _B)
    cont = x[:, _NF:].T.reshape(_NC_FEAT * _B)
    l_flat = L.reshape(_VOCAB)
    w_rm = _relayout(W.T)
    bias16 = jnp.broadcast_to(bias, (_LANES,))
    return _fm_sc(idx, cont, w_rm, l_flat, bias16)


# speedup vs baseline: 1.1954x; 1.1954x over previous
import functools

import jax
import jax.numpy as jnp
from jax import lax
from jax.experimental import pallas as pl
from jax.experimental.pallas import tpu as pltpu
from jax.experimental.pallas import tpu_sc as plsc

_VOCAB = 1000000
_VOCAB_PAD = 1000064
_EMB = 16
_B = 16384
_NF = 9
_NC_FEAT = 3
_LANES = 16
_CHL = 1664
_TC_GRID = _VOCAB_PAD // _CHL

_info = plsc.get_sparse_core_info()
_NW = _info.num_cores * _info.num_subcores
_BPW = _B // _NW
_CHUNKS = _BPW // _LANES

_mesh = plsc.VectorSubcoreMesh(core_axis_name="c", subcore_axis_name="s")


_N_CHUNKS = _VOCAB_PAD // _CHL
_MAX_CHUNKS_PER_W = -(-_N_CHUNKS // _NW)


@functools.partial(
    pl.kernel,
    mesh=_mesh,
    out_type=jax.ShapeDtypeStruct((_VOCAB_PAD, _EMB), jnp.float32),
    compiler_params=pltpu.CompilerParams(
        needs_layout_passes=False, use_tc_tiling_on_sc=False),
    scratch_types=[
        pltpu.VMEM((_EMB, _CHL), jnp.float32),
        pltpu.VMEM((_CHL, _EMB), jnp.float32),
    ],
)
def _relayout_sc(wt_hbm, out_hbm, in_v, out_v):
    wid = lax.axis_index("s") * _info.num_cores + lax.axis_index("c")
    lane = lax.iota(jnp.int32, _LANES)

    def chunk_body(i, _):
        cid = i * _NW + wid

        @pl.when(cid < _N_CHUNKS)
        def _():
            base = cid * _CHL
            pltpu.sync_copy(wt_hbm.at[:, pl.ds(base, _CHL)], in_v)

            def group_body(g, _):
                c0 = g * _LANES
                rows = c0 + lane
                for e in range(_EMB):
                    vals = in_v[e, pl.ds(c0, _LANES)]
                    plsc.store_scatter(
                        out_v, [rows, jnp.full((_LANES,), e, jnp.int32)], vals)
                return ()

            lax.fori_loop(0, _CHL // _LANES, group_body, (), unroll=False)
            pltpu.sync_copy(out_v, out_hbm.at[pl.ds(base, _CHL), :])
        return ()

    lax.fori_loop(0, _MAX_CHUNKS_PER_W, chunk_body, (), unroll=False)


def _relayout(w_t):
    w_p = jnp.pad(w_t, ((0, 0), (0, _VOCAB_PAD - _VOCAB)))
    return _relayout_sc(w_p)


@functools.partial(
    pl.kernel,
    mesh=_mesh,
    out_type=jax.ShapeDtypeStruct((_B,), jnp.float32),
    compiler_params=pltpu.CompilerParams(
        needs_layout_passes=False, use_tc_tiling_on_sc=False),
    scratch_types=(
        [pltpu.VMEM((_BPW,), jnp.int32) for _ in range(_NF)]
        + [pltpu.VMEM((_NF, _BPW, _EMB), jnp.float32)]
        + [pltpu.VMEM((_BPW,), jnp.float32) for _ in range(_NF)]
        + [pltpu.VMEM((_BPW,), jnp.float32) for _ in range(_NC_FEAT)]
        + [
            pltpu.VMEM((_BPW,), jnp.float32),
            pltpu.VMEM((_LANES,), jnp.float32),
            pltpu.VMEM((_LANES, _LANES), jnp.float32),
            pltpu.SemaphoreType.DMA,
        ]
    ),
)
def _fm_sc(idx_hbm, cont_hbm, w_hbm, l_hbm, bias_hbm, out_hbm, *scratch):
    idx_vs = scratch[:_NF]
    rows_v = scratch[_NF]
    lrows_vs = scratch[_NF + 1:2 * _NF + 1]
    cont_vs = scratch[2 * _NF + 1:2 * _NF + 1 + _NC_FEAT]
    out_v, bias_v, tbuf, sem = scratch[2 * _NF + 1 + _NC_FEAT:]

    wid = lax.axis_index("s") * _info.num_cores + lax.axis_index("c")
    base = wid * _BPW

    for j in range(_NF):
        pltpu.sync_copy(idx_hbm.at[pl.ds(j * _B + base, _BPW)], idx_vs[j])
    for k in range(_NC_FEAT):
        pltpu.sync_copy(cont_hbm.at[pl.ds(k * _B + base, _BPW)], cont_vs[k])
    pltpu.sync_copy(bias_hbm, bias_v)

    copies = []
    for j in range(_NF):
        copies.append(pltpu.async_copy(w_hbm.at[idx_vs[j]], rows_v.at[j], sem))
        copies.append(pltpu.async_copy(l_hbm.at[idx_vs[j]], lrows_vs[j], sem))
    for c in copies:
        c.wait()

    lane = lax.iota(jnp.int32, _LANES)
    bias_vec = bias_v[...]

    def chunk_body(c, _):
        row0 = c * _LANES
        for r in range(_LANES):
            row = row0 + r
            e = rows_v[0, row]
            s = e
            ss = e * e
            for j in range(1, _NF):
                e = rows_v[j, row]
                s = s + e
                ss = ss + e * e
            tbuf[r] = s * s - ss
        res = jnp.zeros((_LANES,), jnp.float32)
        for dd in range(_LANES):
            col = plsc.load_gather(
                tbuf, [lane, jnp.full((_LANES,), dd, jnp.int32)])
            res = res + col
        lin = bias_vec
        for j in range(_NF - _NC_FEAT):
            lin = lin + lrows_vs[j][pl.ds(row0, _LANES)]
        for k in range(_NC_FEAT):
            lin = lin + (lrows_vs[_NF - _NC_FEAT + k][pl.ds(row0, _LANES)]
                         * cont_vs[k][pl.ds(row0, _LANES)])
        z = lin + 0.5 * res
        out_v[pl.ds(row0, _LANES)] = 1.0 / (1.0 + jnp.exp(-z))
        return ()

    lax.fori_loop(0, _CHUNKS, chunk_body, (), unroll=False)

    pltpu.sync_copy(out_v, out_hbm.at[pl.ds(base, _BPW)])


def kernel(x, W, L, bias):
    idx = x[:, :_NF].astype(jnp.int32).T.reshape(_NF * _B)
    cont = x[:, _NF:].T.reshape(_NC_FEAT * _B)
    l_flat = L.reshape(_VOCAB)
    w_rm = _relayout(W.T)
    bias16 = jnp.broadcast_to(bias, (_LANES,))
    return _fm_sc(idx, cont, w_rm, l_flat, bias16)


# speedup vs baseline: 3.3646x; 2.8146x over previous
import functools

import jax
import jax.numpy as jnp
from jax import lax
from jax.experimental import pallas as pl
from jax.experimental.pallas import tpu as pltpu
from jax.experimental.pallas import tpu_sc as plsc

_VOCAB = 1000000
_EMB = 16
_B = 16384
_NF = 9
_NC_FEAT = 3
_LANES = 16

_info = plsc.get_sparse_core_info()
_NW = _info.num_cores * _info.num_subcores
_BPW = _B // _NW
_CHUNKS = _BPW // _LANES

_mesh = plsc.VectorSubcoreMesh(core_axis_name="c", subcore_axis_name="s")


@functools.partial(
    pl.kernel,
    mesh=_mesh,
    out_type=jax.ShapeDtypeStruct((_B,), jnp.float32),
    compiler_params=pltpu.CompilerParams(
        needs_layout_passes=False, use_tc_tiling_on_sc=False),
    scratch_types=(
        [pltpu.VMEM((_BPW,), jnp.int32) for _ in range(_NF)]
        + [pltpu.VMEM((_NF, _BPW, _EMB), jnp.float32)]
        + [pltpu.VMEM((_BPW,), jnp.float32) for _ in range(_NF)]
        + [pltpu.VMEM((_BPW,), jnp.float32) for _ in range(_NC_FEAT)]
        + [
            pltpu.VMEM((_BPW,), jnp.float32),
            pltpu.VMEM((_LANES,), jnp.float32),
            pltpu.VMEM((_LANES, _LANES), jnp.float32),
            pltpu.SemaphoreType.DMA,
        ]
    ),
)
def _fm_sc(idx_hbm, cont_hbm, w_hbm, l_hbm, bias_hbm, out_hbm, *scratch):
    idx_vs = scratch[:_NF]
    rows_v = scratch[_NF]
    lrows_vs = scratch[_NF + 1:2 * _NF + 1]
    cont_vs = scratch[2 * _NF + 1:2 * _NF + 1 + _NC_FEAT]
    out_v, bias_v, tbuf, sem = scratch[2 * _NF + 1 + _NC_FEAT:]

    wid = lax.axis_index("s") * _info.num_cores + lax.axis_index("c")
    base = wid * _BPW

    for j in range(_NF):
        pltpu.sync_copy(idx_hbm.at[pl.ds(j * _B + base, _BPW)], idx_vs[j])
    for k in range(_NC_FEAT):
        pltpu.sync_copy(cont_hbm.at[pl.ds(k * _B + base, _BPW)], cont_vs[k])
    pltpu.sync_copy(bias_hbm, bias_v)

    copies = []
    l_view = l_hbm.at[0]
    for j in range(_NF):
        copies.append(pltpu.async_copy(w_hbm.at[idx_vs[j]], rows_v.at[j], sem))
        copies.append(pltpu.async_copy(l_view.at[idx_vs[j]], lrows_vs[j], sem))
    for c in copies:
        c.wait()

    lane = lax.iota(jnp.int32, _LANES)
    bias_vec = bias_v[...]

    def chunk_body(c, _):
        row0 = c * _LANES
        for r in range(_LANES):
            row = row0 + r
            e = rows_v[0, row]
            s = e
            ss = e * e
            for j in range(1, _NF):
                e = rows_v[j, row]
                s = s + e
                ss = ss + e * e
            tbuf[r] = s * s - ss
        res = jnp.zeros((_LANES,), jnp.float32)
        for dd in range(_LANES):
            col = plsc.load_gather(
                tbuf, [lane, jnp.full((_LANES,), dd, jnp.int32)])
            res = res + col
        lin = bias_vec
        for j in range(_NF - _NC_FEAT):
            lin = lin + lrows_vs[j][pl.ds(row0, _LANES)]
        for k in range(_NC_FEAT):
            lin = lin + (lrows_vs[_NF - _NC_FEAT + k][pl.ds(row0, _LANES)]
                         * cont_vs[k][pl.ds(row0, _LANES)])
        z = lin + 0.5 * res
        out_v[pl.ds(row0, _LANES)] = 1.0 / (1.0 + jnp.exp(-z))
        return ()

    lax.fori_loop(0, _CHUNKS, chunk_body, (), unroll=False)

    pltpu.sync_copy(out_v, out_hbm.at[pl.ds(base, _BPW)])


def kernel(x, W, L, bias):
    idx = x[:, :_NF].astype(jnp.int32).T.reshape(_NF * _B)
    cont = x[:, _NF:].T.reshape(_NC_FEAT * _B)
    l_pad = jnp.pad(L.T, ((0, 0), (0, 64)))
    bias16 = jnp.broadcast_to(bias, (_LANES,))
    return _fm_sc(idx, cont, W, l_pad, bias16)


# speedup vs baseline: 3.4070x; 1.0126x over previous
import functools

import jax
import jax.numpy as jnp
from jax import lax
from jax.experimental import pallas as pl
from jax.experimental.pallas import tpu as pltpu
from jax.experimental.pallas import tpu_sc as plsc

_VOCAB = 1000000
_EMB = 16
_B = 16384
_NF = 9
_NC_FEAT = 3
_LANES = 16

_info = plsc.get_sparse_core_info()
_NW = _info.num_cores * _info.num_subcores
_BPW = _B // _NW
_CHUNKS = _BPW // _LANES

_mesh = plsc.VectorSubcoreMesh(core_axis_name="c", subcore_axis_name="s")


@functools.partial(
    pl.kernel,
    mesh=_mesh,
    out_type=jax.ShapeDtypeStruct((_B,), jnp.float32),
    compiler_params=pltpu.CompilerParams(
        needs_layout_passes=False, use_tc_tiling_on_sc=False),
    scratch_types=(
        [pltpu.VMEM((_BPW,), jnp.int32) for _ in range(_NF)]
        + [pltpu.VMEM((_NF, _BPW, _EMB), jnp.float32)]
        + [pltpu.VMEM((_BPW,), jnp.float32) for _ in range(_NF)]
        + [pltpu.VMEM((_BPW,), jnp.float32) for _ in range(_NC_FEAT)]
        + [
            pltpu.VMEM((_BPW,), jnp.float32),
            pltpu.VMEM((_LANES,), jnp.float32),
            pltpu.VMEM((_LANES, _LANES), jnp.float32),
            pltpu.SemaphoreType.DMA,
        ]
    ),
)
def _fm_sc(idx_hbm, cont_hbm, w_hbm, l_hbm, bias_hbm, out_hbm, *scratch):
    idx_vs = scratch[:_NF]
    rows_v = scratch[_NF]
    lrows_vs = scratch[_NF + 1:2 * _NF + 1]
    cont_vs = scratch[2 * _NF + 1:2 * _NF + 1 + _NC_FEAT]
    out_v, bias_v, tbuf, sem = scratch[2 * _NF + 1 + _NC_FEAT:]

    wid = lax.axis_index("s") * _info.num_cores + lax.axis_index("c")
    base = wid * _BPW

    stage = []
    for j in range(_NF):
        stage.append(pltpu.async_copy(
            idx_hbm.at[pl.ds(j * _B + base, _BPW)], idx_vs[j], sem))
    for k in range(_NC_FEAT):
        stage.append(pltpu.async_copy(
            cont_hbm.at[pl.ds(k * _B + base, _BPW)], cont_vs[k], sem))
    stage.append(pltpu.async_copy(bias_hbm, bias_v, sem))
    for c in stage:
        c.wait()

    copies = []
    l_view = l_hbm.at[0]
    for j in range(_NF):
        copies.append(pltpu.async_copy(w_hbm.at[idx_vs[j]], rows_v.at[j], sem))
        copies.append(pltpu.async_copy(l_view.at[idx_vs[j]], lrows_vs[j], sem))
    for c in copies:
        c.wait()

    lane = lax.iota(jnp.int32, _LANES)
    bias_vec = bias_v[...]

    def chunk_body(c, _):
        row0 = c * _LANES
        for r in range(_LANES):
            row = row0 + r
            e = rows_v[0, row]
            s = e
            ss = e * e
            for j in range(1, _NF):
                e = rows_v[j, row]
                s = s + e
                ss = ss + e * e
            tbuf[r] = s * s - ss
        res = jnp.zeros((_LANES,), jnp.float32)
        for dd in range(_LANES):
            col = plsc.load_gather(
                tbuf, [lane, jnp.full((_LANES,), dd, jnp.int32)])
            res = res + col
        lin = bias_vec
        for j in range(_NF - _NC_FEAT):
            lin = lin + lrows_vs[j][pl.ds(row0, _LANES)]
        for k in range(_NC_FEAT):
            lin = lin + (lrows_vs[_NF - _NC_FEAT + k][pl.ds(row0, _LANES)]
                         * cont_vs[k][pl.ds(row0, _LANES)])
        z = lin + 0.5 * res
        out_v[pl.ds(row0, _LANES)] = 1.0 / (1.0 + jnp.exp(-z))
        return ()

    lax.fori_loop(0, _CHUNKS, chunk_body, (), unroll=False)

    pltpu.sync_copy(out_v, out_hbm.at[pl.ds(base, _BPW)])


def kernel(x, W, L, bias):
    idx = x[:, :_NF].astype(jnp.int32).T.reshape(_NF * _B)
    cont = x[:, _NF:].T.reshape(_NC_FEAT * _B)
    l_pad = jnp.pad(L.T, ((0, 0), (0, 64)))
    bias16 = jnp.broadcast_to(bias, (_LANES,))
    return _fm_sc(idx, cont, W, l_pad, bias16)
